# baseline (speedup 1.0000x reference)
import functools

import jax
import jax.numpy as jnp
from jax import lax
from jax.experimental import pallas as pl
from jax.experimental.pallas import tpu as pltpu
from jax.experimental.pallas import tpu_sc as plsc

B = 4096
F = 100
FP = 104
D = 128
NW = 32
BPW = B // NW
FVP = 128
NBUF = 4


def _sc_body(idx_hbm, val_hbm, fld_hbm, emb_hbm, x_out, cnt_out,
             idx_v, val_v, fld_v, rows_v, acc_v, cnt_v, *sems):
    wid = lax.axis_index("s") * 2 + lax.axis_index("c")
    base = wid * BPW
    pltpu.sync_copy(idx_hbm.at[pl.ds(base, BPW)], idx_v)
    pltpu.sync_copy(val_hbm.at[pl.ds(base, BPW)], val_v)
    pltpu.sync_copy(fld_hbm.at[pl.ds(base, BPW)], fld_v)

    lanes = lax.iota(jnp.int32, 16)
    zeros16 = jnp.zeros((16,), jnp.float32)
    ones16 = jnp.ones((16,), jnp.float32)

    def issue(b, j):
        return pltpu.async_copy(
            emb_hbm.at[idx_v.at[b].at[pl.ds(0, F)]], rows_v.at[j], sems[j])

    for j in range(NBUF):
        issue(j, j)

    def group(g, carry):
        for j in range(NBUF):
            b = g * NBUF + j
            pltpu.make_async_copy(
                emb_hbm.at[idx_v.at[b].at[pl.ds(0, F)]], rows_v.at[j], sems[j]).wait()
            for c in range(8):
                cnt_v[b, 16 * c:16 * (c + 1)] = zeros16
            bfull = jnp.full((16,), b, jnp.int32)
            for k in range(7):
                pos = jnp.int32(16 * k) + lanes
                fidx = jnp.minimum(pos, jnp.int32(F - 1))
                fv = plsc.load_gather(fld_v, [bfull, fidx])
                plsc.addupdate_scatter(cnt_v, [bfull, fv], ones16,
                                       mask=pos < F)
            def fbody(f, accs):
                w = plsc.load_gather(
                    val_v, [bfull, jnp.full((16,), f, jnp.int32)])
                out = []
                for g2 in range(4):
                    v = plsc.bitcast(rows_v[j, f, 32 * g2:32 * (g2 + 1)],
                                     jnp.int32)
                    lo = plsc.bitcast(lax.shift_left(v, jnp.int32(16)),
                                      jnp.float32)
                    hi = plsc.bitcast(
                        lax.bitwise_and(v, jnp.int32(-65536)), jnp.float32)
                    out.append(accs[2 * g2] + w * lo)
                    out.append(accs[2 * g2 + 1] + w * hi)
                return tuple(out)

            accs = lax.fori_loop(0, F, fbody, (zeros16,) * 8, unroll=5)
            for c in range(8):
                acc_v[b, 16 * c:16 * (c + 1)] = accs[c]
            @pl.when(b + NBUF < BPW)
            def _():
                issue(b + NBUF, j)
        return carry

    lax.fori_loop(0, BPW // NBUF, group, 0)
    pltpu.sync_copy(acc_v, x_out.at[pl.ds(base, BPW)])
    pltpu.sync_copy(cnt_v, cnt_out.at[pl.ds(base, BPW)])


_sc_pooling = functools.partial(
    pl.kernel,
    mesh=plsc.VectorSubcoreMesh(core_axis_name="c", subcore_axis_name="s"),
    compiler_params=pltpu.CompilerParams(needs_layout_passes=False,
                                         use_tc_tiling_on_sc=False),
    out_type=(jax.ShapeDtypeStruct((B, D), jnp.float32),
              jax.ShapeDtypeStruct((B, FVP), jnp.float32)),
    scratch_types=[
        pltpu.VMEM((BPW, FP), jnp.int32),
        pltpu.VMEM((BPW, F), jnp.float32),
        pltpu.VMEM((BPW, F), jnp.int32),
        pltpu.VMEM((NBUF, F, D), jnp.bfloat16),
        pltpu.VMEM((BPW, D), jnp.float32),
        pltpu.VMEM((BPW, FVP), jnp.float32),
    ] + [pltpu.SemaphoreType.DMA] * NBUF,
)(_sc_body)


BT = 512


def _mlp_body(x_ref, cnt_ref, doc_ref, femb_ref, Wd_ref, bd_ref,
              W1_ref, b1_ref, W2_ref, b2_ref, Wo_ref, bo_ref, o_ref):
    f32 = jnp.float32
    W1 = W1_ref[...]
    fw = jnp.dot(femb_ref[...], W1[D:2 * D, :], preferred_element_type=f32)
    x2 = jnp.maximum(
        jnp.dot(doc_ref[...], Wd_ref[...], preferred_element_type=f32)
        + bd_ref[...], 0.0)
    h1 = (jnp.dot(x_ref[...], W1[:D, :], preferred_element_type=f32)
          + jnp.dot(cnt_ref[...], fw, preferred_element_type=f32)
          + jnp.dot(x2, W1[2 * D:, :], preferred_element_type=f32)
          + b1_ref[...])
    h1 = jnp.maximum(h1, 0.0)
    h2 = jnp.maximum(
        jnp.dot(h1, W2_ref[...], preferred_element_type=f32) + b2_ref[...],
        0.0)
    o_ref[...] = jnp.dot(h2, Wo_ref[...], preferred_element_type=f32) + bo_ref[...]


def _mlp(x, cnt, doc, femb_p, Wd, bd, W1, b1, W2, b2, Wo, bo):
    M1 = W1.shape[1]
    M2 = W2.shape[1]
    grid = (B // BT,)
    return pl.pallas_call(
        _mlp_body,
        grid=grid,
        in_specs=[
            pl.BlockSpec((BT, D), lambda i: (i, 0)),
            pl.BlockSpec((BT, FVP), lambda i: (i, 0)),
            pl.BlockSpec((BT, D), lambda i: (i, 0)),
            pl.BlockSpec((FVP, D), lambda i: (0, 0)),
            pl.BlockSpec((D, D), lambda i: (0, 0)),
            pl.BlockSpec((D,), lambda i: (0,)),
            pl.BlockSpec((3 * D, M1), lambda i: (0, 0)),
            pl.BlockSpec((M1,), lambda i: (0,)),
            pl.BlockSpec((M1, M2), lambda i: (0, 0)),
            pl.BlockSpec((M2,), lambda i: (0,)),
            pl.BlockSpec((M2, 1), lambda i: (0, 0)),
            pl.BlockSpec((1,), lambda i: (0,)),
        ],
        out_specs=pl.BlockSpec((BT, 1), lambda i: (i, 0)),
        out_shape=jax.ShapeDtypeStruct((B, 1), jnp.float32),
    )(x, cnt, doc, femb_p, Wd, bd, W1, b1, W2, b2, Wo, bo)


CROWS = 1024


def _cast_body(x_ref, o_ref):
    o_ref[...] = x_ref[...].astype(jnp.bfloat16).reshape(CROWS * D)


def _cast_table(emb):
    V = emb.shape[0]
    nblk = (V + CROWS - 1) // CROWS
    flat = pl.pallas_call(
        _cast_body,
        grid=(nblk,),
        in_specs=[pl.BlockSpec((CROWS, D), lambda i: (i, 0))],
        out_specs=pl.BlockSpec((CROWS * D,), lambda i: (i,)),
        out_shape=jax.ShapeDtypeStruct((nblk * CROWS * D,), jnp.bfloat16),
    )(emb)
    return flat.reshape(nblk * CROWS, D)


def kernel(index, value, field, doc_emb, emb, femb, Wd, bd,
           W1, b1, W2, b2, Wo, bo):
    index = index.astype(jnp.int32)
    field = field.astype(jnp.int32)
    idx_p = jnp.pad(index, ((0, 0), (0, FP - F)))
    emb_bf = _cast_table(emb)
    x_emb, cnt = _sc_pooling(idx_p, value, field, emb_bf)
    gg, hh, kk = jnp.meshgrid(jnp.arange(4), jnp.arange(2), jnp.arange(16),
                              indexing="ij")
    perm = (32 * gg + 2 * kk + hh).reshape(D)
    W1 = jnp.concatenate([W1[:D][perm], W1[D:]], axis=0)
    femb_p = jnp.zeros((FVP, D), femb.dtype).at[:femb.shape[0]].set(femb)
    out = _mlp(x_emb, cnt, doc_emb, femb_p, Wd, bd, W1, b1, W2, b2, Wo, bo)
    return jnp.squeeze(out, -1)


# speedup vs baseline: 2.1584x; 2.1584x over previous
import functools

import jax
import jax.numpy as jnp
from jax import lax
from jax.experimental import pallas as pl
from jax.experimental.pallas import tpu as pltpu
from jax.experimental.pallas import tpu_sc as plsc

B = 4096
F = 100
FP = 104
D = 128
NW = 32
BPW = B // NW
FVP = 128
NBUF = 4


def _sc_body(idx_hbm, val_hbm, fld_hbm, emb_hbm, x_out, cnt_out,
             idx_v, val_v, fld_v, rows_v, acc_v, cnt_v, *sems):
    wid = lax.axis_index("s") * 2 + lax.axis_index("c")
    base = wid * BPW
    pltpu.sync_copy(idx_hbm.at[pl.ds(base, BPW)], idx_v)
    pltpu.sync_copy(val_hbm.at[pl.ds(base, BPW)], val_v)
    pltpu.sync_copy(fld_hbm.at[pl.ds(base, BPW)], fld_v)

    lanes = lax.iota(jnp.int32, 16)
    zeros16 = jnp.zeros((16,), jnp.float32)
    ones16 = jnp.ones((16,), jnp.float32)

    def issue(b, j):
        return pltpu.async_copy(
            emb_hbm.at[idx_v.at[b].at[pl.ds(0, F)]], rows_v.at[j], sems[j])

    for j in range(NBUF):
        issue(j, j)

    def group(g, carry):
        for j in range(NBUF):
            b = g * NBUF + j
            pltpu.make_async_copy(
                emb_hbm.at[idx_v.at[b].at[pl.ds(0, F)]], rows_v.at[j], sems[j]).wait()
            for c in range(8):
                cnt_v[b, 16 * c:16 * (c + 1)] = zeros16
            bfull = jnp.full((16,), b, jnp.int32)
            for k in range(7):
                pos = jnp.int32(16 * k) + lanes
                fidx = jnp.minimum(pos, jnp.int32(F - 1))
                fv = plsc.load_gather(fld_v, [bfull, fidx])
                plsc.addupdate_scatter(cnt_v, [bfull, fv], ones16,
                                       mask=pos < F)
            def fbody(f, accs):
                w = plsc.load_gather(
                    val_v, [bfull, jnp.full((16,), f, jnp.int32)])
                return tuple(
                    accs[c] + w * rows_v[j, f, 16 * c:16 * (c + 1)]
                    for c in range(8))

            accs = lax.fori_loop(0, F, fbody, (zeros16,) * 8, unroll=5)
            for c in range(8):
                acc_v[b, 16 * c:16 * (c + 1)] = accs[c]
            @pl.when(b + NBUF < BPW)
            def _():
                issue(b + NBUF, j)
        return carry

    lax.fori_loop(0, BPW // NBUF, group, 0)
    pltpu.sync_copy(acc_v, x_out.at[pl.ds(base, BPW)])
    pltpu.sync_copy(cnt_v, cnt_out.at[pl.ds(base, BPW)])


_sc_pooling = functools.partial(
    pl.kernel,
    mesh=plsc.VectorSubcoreMesh(core_axis_name="c", subcore_axis_name="s"),
    compiler_params=pltpu.CompilerParams(needs_layout_passes=False,
                                         use_tc_tiling_on_sc=False),
    out_type=(jax.ShapeDtypeStruct((B, D), jnp.float32),
              jax.ShapeDtypeStruct((B, FVP), jnp.float32)),
    scratch_types=[
        pltpu.VMEM((BPW, FP), jnp.int32),
        pltpu.VMEM((BPW, F), jnp.float32),
        pltpu.VMEM((BPW, F), jnp.int32),
        pltpu.VMEM((NBUF, F, D), jnp.float32),
        pltpu.VMEM((BPW, D), jnp.float32),
        pltpu.VMEM((BPW, FVP), jnp.float32),
    ] + [pltpu.SemaphoreType.DMA] * NBUF,
)(_sc_body)


BT = 512


def _mlp_body(x_ref, cnt_ref, doc_ref, femb_ref, Wd_ref, bd_ref,
              W1_ref, b1_ref, W2_ref, b2_ref, Wo_ref, bo_ref, o_ref):
    f32 = jnp.float32
    W1 = W1_ref[...]
    fw = jnp.dot(femb_ref[...], W1[D:2 * D, :], preferred_element_type=f32)
    x2 = jnp.maximum(
        jnp.dot(doc_ref[...], Wd_ref[...], preferred_element_type=f32)
        + bd_ref[...], 0.0)
    h1 = (jnp.dot(x_ref[...], W1[:D, :], preferred_element_type=f32)
          + jnp.dot(cnt_ref[...], fw, preferred_element_type=f32)
          + jnp.dot(x2, W1[2 * D:, :], preferred_element_type=f32)
          + b1_ref[...])
    h1 = jnp.maximum(h1, 0.0)
    h2 = jnp.maximum(
        jnp.dot(h1, W2_ref[...], preferred_element_type=f32) + b2_ref[...],
        0.0)
    o_ref[...] = jnp.dot(h2, Wo_ref[...], preferred_element_type=f32) + bo_ref[...]


def _mlp(x, cnt, doc, femb_p, Wd, bd, W1, b1, W2, b2, Wo, bo):
    M1 = W1.shape[1]
    M2 = W2.shape[1]
    grid = (B // BT,)
    return pl.pallas_call(
        _mlp_body,
        grid=grid,
        in_specs=[
            pl.BlockSpec((BT, D), lambda i: (i, 0)),
            pl.BlockSpec((BT, FVP), lambda i: (i, 0)),
            pl.BlockSpec((BT, D), lambda i: (i, 0)),
            pl.BlockSpec((FVP, D), lambda i: (0, 0)),
            pl.BlockSpec((D, D), lambda i: (0, 0)),
            pl.BlockSpec((D,), lambda i: (0,)),
            pl.BlockSpec((3 * D, M1), lambda i: (0, 0)),
            pl.BlockSpec((M1,), lambda i: (0,)),
            pl.BlockSpec((M1, M2), lambda i: (0, 0)),
            pl.BlockSpec((M2,), lambda i: (0,)),
            pl.BlockSpec((M2, 1), lambda i: (0, 0)),
            pl.BlockSpec((1,), lambda i: (0,)),
        ],
        out_specs=pl.BlockSpec((BT, 1), lambda i: (i, 0)),
        out_shape=jax.ShapeDtypeStruct((B, 1), jnp.float32),
    )(x, cnt, doc, femb_p, Wd, bd, W1, b1, W2, b2, Wo, bo)


CROWS = 1024


def _cast_body(x_ref, o_ref):
    o_ref[...] = x_ref[...].astype(jnp.bfloat16).reshape(CROWS * D)


def _cast_table(emb):
    V = emb.shape[0]
    nblk = (V + CROWS - 1) // CROWS
    flat = pl.pallas_call(
        _cast_body,
        grid=(nblk,),
        in_specs=[pl.BlockSpec((CROWS, D), lambda i: (i, 0))],
        out_specs=pl.BlockSpec((CROWS * D,), lambda i: (i,)),
        out_shape=jax.ShapeDtypeStruct((nblk * CROWS * D,), jnp.bfloat16),
    )(emb)
    return flat.reshape(nblk * CROWS, D)


def kernel(index, value, field, doc_emb, emb, femb, Wd, bd,
           W1, b1, W2, b2, Wo, bo):
    index = index.astype(jnp.int32)
    field = field.astype(jnp.int32)
    idx_p = jnp.pad(index, ((0, 0), (0, FP - F)))
    x_emb, cnt = _sc_pooling(idx_p, value, field, emb)
    femb_p = jnp.zeros((FVP, D), femb.dtype).at[:femb.shape[0]].set(femb)
    out = _mlp(x_emb, cnt, doc_emb, femb_p, Wd, bd, W1, b1, W2, b2, Wo, bo)
    return jnp.squeeze(out, -1)


# speedup vs baseline: 2.1600x; 1.0007x over previous
import functools

import jax
import jax.numpy as jnp
from jax import lax
from jax.experimental import pallas as pl
from jax.experimental.pallas import tpu as pltpu
from jax.experimental.pallas import tpu_sc as plsc

B = 4096
F = 100
FP = 104
D = 128
NW = 32
BPW = B // NW
FVP = 128
NBUF = 4


def _sc_body(idx_hbm, val_hbm, fld_hbm, emb_hbm, x_out, cnt_out,
             idx_v, val_v, fld_v, rows_v, acc_v, cnt_v, *sems):
    wid = lax.axis_index("s") * 2 + lax.axis_index("c")
    base = wid * BPW
    pltpu.sync_copy(idx_hbm.at[pl.ds(base, BPW)], idx_v)
    pltpu.sync_copy(val_hbm.at[pl.ds(base, BPW)], val_v)
    pltpu.sync_copy(fld_hbm.at[pl.ds(base, BPW)], fld_v)

    lanes = lax.iota(jnp.int32, 16)
    zeros16 = jnp.zeros((16,), jnp.float32)
    ones16 = jnp.ones((16,), jnp.float32)

    def issue(b, j):
        return pltpu.async_copy(
            emb_hbm.at[idx_v.at[b].at[pl.ds(0, F)]], rows_v.at[j], sems[j])

    for j in range(NBUF):
        issue(j, j)

    def group(g, carry):
        for j in range(NBUF):
            b = g * NBUF + j
            pltpu.make_async_copy(
                emb_hbm.at[idx_v.at[b].at[pl.ds(0, F)]], rows_v.at[j], sems[j]).wait()
            for c in range(8):
                cnt_v[b, 16 * c:16 * (c + 1)] = zeros16
            bfull = jnp.full((16,), b, jnp.int32)
            for k in range(7):
                pos = jnp.int32(16 * k) + lanes
                fidx = jnp.minimum(pos, jnp.int32(F - 1))
                fv = plsc.load_gather(fld_v, [bfull, fidx])
                plsc.addupdate_scatter(cnt_v, [bfull, fv], ones16,
                                       mask=pos < F)
            def fbody(f, accs):
                w = plsc.load_gather(
                    val_v, [bfull, jnp.full((16,), f, jnp.int32)])
                return tuple(
                    accs[c] + w * rows_v[j, f, 16 * c:16 * (c + 1)]
                    for c in range(8))

            accs = lax.fori_loop(0, F, fbody, (zeros16,) * 8, unroll=5)
            for c in range(8):
                acc_v[b, 16 * c:16 * (c + 1)] = accs[c]
            @pl.when(b + NBUF < BPW)
            def _():
                issue(b + NBUF, j)
        return carry

    lax.fori_loop(0, BPW // NBUF, group, 0)
    pltpu.sync_copy(acc_v, x_out.at[pl.ds(base, BPW)])
    pltpu.sync_copy(cnt_v, cnt_out.at[pl.ds(base, BPW)])


_sc_pooling = functools.partial(
    pl.kernel,
    mesh=plsc.VectorSubcoreMesh(core_axis_name="c", subcore_axis_name="s"),
    compiler_params=pltpu.CompilerParams(needs_layout_passes=False,
                                         use_tc_tiling_on_sc=False),
    out_type=(jax.ShapeDtypeStruct((B, D), jnp.float32),
              jax.ShapeDtypeStruct((B, FVP), jnp.float32)),
    scratch_types=[
        pltpu.VMEM((BPW, FP), jnp.int32),
        pltpu.VMEM((BPW, F), jnp.float32),
        pltpu.VMEM((BPW, F), jnp.int32),
        pltpu.VMEM((NBUF, F, D), jnp.float32),
        pltpu.VMEM((BPW, D), jnp.float32),
        pltpu.VMEM((BPW, FVP), jnp.float32),
    ] + [pltpu.SemaphoreType.DMA] * NBUF,
)(_sc_body)


BT = 512


def _mlp_body(x_ref, cnt_ref, doc_ref, femb_ref, Wd_ref, bd_ref,
              W1_ref, b1_ref, W2_ref, b2_ref, Wo_ref, bo_ref, o_ref):
    f32 = jnp.float32
    W1 = W1_ref[...]
    fw = jnp.dot(femb_ref[...], W1[D:2 * D, :], preferred_element_type=f32)
    x2 = jnp.maximum(
        jnp.dot(doc_ref[...], Wd_ref[...], preferred_element_type=f32)
        + bd_ref[...], 0.0)
    h1 = (jnp.dot(x_ref[...], W1[:D, :], preferred_element_type=f32)
          + jnp.dot(cnt_ref[...], fw, preferred_element_type=f32)
          + jnp.dot(x2, W1[2 * D:, :], preferred_element_type=f32)
          + b1_ref[...])
    h1 = jnp.maximum(h1, 0.0)
    h2 = jnp.maximum(
        jnp.dot(h1, W2_ref[...], preferred_element_type=f32) + b2_ref[...],
        0.0)
    o_ref[...] = jnp.dot(h2, Wo_ref[...], preferred_element_type=f32) + bo_ref[...]


def _mlp(x, cnt, doc, femb_p, Wd, bd, W1, b1, W2, b2, Wo, bo):
    M1 = W1.shape[1]
    M2 = W2.shape[1]
    grid = (B // BT,)
    return pl.pallas_call(
        _mlp_body,
        grid=grid,
        in_specs=[
            pl.BlockSpec((BT, D), lambda i: (i, 0)),
            pl.BlockSpec((BT, FVP), lambda i: (i, 0)),
            pl.BlockSpec((BT, D), lambda i: (i, 0)),
            pl.BlockSpec((FVP, D), lambda i: (0, 0)),
            pl.BlockSpec((D, D), lambda i: (0, 0)),
            pl.BlockSpec((D,), lambda i: (0,)),
            pl.BlockSpec((3 * D, M1), lambda i: (0, 0)),
            pl.BlockSpec((M1,), lambda i: (0,)),
            pl.BlockSpec((M1, M2), lambda i: (0, 0)),
            pl.BlockSpec((M2,), lambda i: (0,)),
            pl.BlockSpec((M2, 1), lambda i: (0, 0)),
            pl.BlockSpec((1,), lambda i: (0,)),
        ],
        out_specs=pl.BlockSpec((BT, 1), lambda i: (i, 0)),
        out_shape=jax.ShapeDtypeStruct((B, 1), jnp.float32),
    )(x, cnt, doc, femb_p, Wd, bd, W1, b1, W2, b2, Wo, bo)


def kernel(index, value, field, doc_emb, emb, femb, Wd, bd,
           W1, b1, W2, b2, Wo, bo):
    index = index.astype(jnp.int32)
    field = field.astype(jnp.int32)
    idx_p = jnp.pad(index, ((0, 0), (0, FP - F)))
    x_emb, cnt = _sc_pooling(idx_p, value, field, emb)
    femb_p = jnp.zeros((FVP, D), femb.dtype).at[:femb.shape[0]].set(femb)
    out = _mlp(x_emb, cnt, doc_emb, femb_p, Wd, bd, W1, b1, W2, b2, Wo, bo)
    return jnp.squeeze(out, -1)
